# baseline (speedup 1.0000x reference)
import functools

import jax
import jax.numpy as jnp
from jax import lax
from jax.experimental import pallas as pl
from jax.experimental.pallas import tpu as pltpu
from jax.experimental.pallas import tpu_sc as plsc

_NUM_CORES = 2
_NUM_SUBCORES = 16
_NW = _NUM_CORES * _NUM_SUBCORES
_BLK = 8
_NBUF = 4
_REP = 2


def kernel(prefix, embedding_weight):
    batch, seq = prefix.shape
    vocab, row_dim = embedding_weight.shape
    bpw = batch // _NW
    nh = bpw // _BLK

    table_rep = jnp.tile(embedding_weight, (_REP, 1))
    rep_off = ((jnp.arange(batch, dtype=jnp.int32) // bpw) % _REP) * vocab
    idx_t = prefix.T + rep_off[None, :]

    mesh = plsc.VectorSubcoreMesh(core_axis_name="c", subcore_axis_name="s")

    @functools.partial(
        pl.kernel,
        mesh=mesh,
        out_type=jax.ShapeDtypeStruct((seq, batch, row_dim), jnp.float32),
        scratch_types=[
            pltpu.VMEM((seq, 128), jnp.int32),
        ] + [pltpu.VMEM((_BLK, row_dim), jnp.float32)] * _NBUF
          + [pltpu.SemaphoreType.DMA] * (2 * _NBUF),
    )
    def sc_gather(table_hbm, idx_hbm, out_hbm, idx_v, *bufs_sems):
        bufs = bufs_sems[:_NBUF]
        gsems = bufs_sems[_NBUF:2 * _NBUF]
        wsems = bufs_sems[2 * _NBUF:]
        wid = lax.axis_index("s") * _NUM_CORES + lax.axis_index("c")
        base = wid * bpw
        pltpu.sync_copy(idx_hbm.at[:, pl.ds((wid // 4) * 128, 128)], idx_v)
        col0 = (wid % 4) * bpw

        n_items = seq * nh

        def g_copy(t, b):
            s, q = t // nh, t % nh
            return pltpu.make_async_copy(
                table_hbm.at[idx_v.at[s, pl.ds(col0 + q * _BLK, _BLK)]],
                bufs[b], gsems[b])

        def w_copy(t, b):
            s, q = t // nh, t % nh
            return pltpu.make_async_copy(
                bufs[b], out_hbm.at[s, pl.ds(base + q * _BLK, _BLK)],
                wsems[b])

        for b in range(_NBUF):
            g_copy(b, b).start()

        @pl.loop(0, n_items - _NBUF, step=_NBUF)
        def _(t):
            for b in range(_NBUF):
                g_copy(t + b, b).wait()
                w_copy(t + b, b).start()
            for b in range(_NBUF):
                w_copy(t + b, b).wait()
                g_copy(t + b + _NBUF, b).start()

        t_last = n_items - _NBUF
        for b in range(_NBUF):
            g_copy(t_last + b, b).wait()
            w_copy(t_last + b, b).start()
        for b in range(_NBUF):
            w_copy(t_last + b, b).wait()

    out_t = sc_gather(table_rep, idx_t)
    return out_t.transpose(1, 0, 2)


# speedup vs baseline: 1.0022x; 1.0022x over previous
import functools

import jax
import jax.numpy as jnp
from jax import lax
from jax.experimental import pallas as pl
from jax.experimental.pallas import tpu as pltpu
from jax.experimental.pallas import tpu_sc as plsc

_NUM_CORES = 2
_NUM_SUBCORES = 16
_NW = _NUM_CORES * _NUM_SUBCORES
_BLK = 8
_NBUF = 4
_REP = 4


def kernel(prefix, embedding_weight):
    batch, seq = prefix.shape
    vocab, row_dim = embedding_weight.shape
    bpw = batch // _NW
    nh = bpw // _BLK

    table_rep = jnp.tile(embedding_weight, (_REP, 1))
    rep_off = ((jnp.arange(batch, dtype=jnp.int32) // bpw) % _REP) * vocab
    idx_t = prefix.T + rep_off[None, :]

    mesh = plsc.VectorSubcoreMesh(core_axis_name="c", subcore_axis_name="s")

    @functools.partial(
        pl.kernel,
        mesh=mesh,
        out_type=jax.ShapeDtypeStruct((seq, batch, row_dim), jnp.float32),
        scratch_types=[
            pltpu.VMEM((seq, 128), jnp.int32),
        ] + [pltpu.VMEM((_BLK, row_dim), jnp.float32)] * _NBUF
          + [pltpu.SemaphoreType.DMA] * (2 * _NBUF),
    )
    def sc_gather(table_hbm, idx_hbm, out_hbm, idx_v, *bufs_sems):
        bufs = bufs_sems[:_NBUF]
        gsems = bufs_sems[_NBUF:2 * _NBUF]
        wsems = bufs_sems[2 * _NBUF:]
        wid = lax.axis_index("s") * _NUM_CORES + lax.axis_index("c")
        base = wid * bpw
        pltpu.sync_copy(idx_hbm.at[:, pl.ds((wid // 4) * 128, 128)], idx_v)
        col0 = (wid % 4) * bpw

        n_items = seq * nh

        def g_copy(t, b):
            s, q = t // nh, t % nh
            return pltpu.make_async_copy(
                table_hbm.at[idx_v.at[s, pl.ds(col0 + q * _BLK, _BLK)]],
                bufs[b], gsems[b])

        def w_copy(t, b):
            s, q = t // nh, t % nh
            return pltpu.make_async_copy(
                bufs[b], out_hbm.at[s, pl.ds(base + q * _BLK, _BLK)],
                wsems[b])

        for b in range(_NBUF):
            g_copy(b, b).start()

        @pl.loop(0, n_items - _NBUF, step=_NBUF)
        def _(t):
            for b in range(_NBUF):
                g_copy(t + b, b).wait()
                w_copy(t + b, b).start()
            for b in range(_NBUF):
                w_copy(t + b, b).wait()
                g_copy(t + b + _NBUF, b).start()

        t_last = n_items - _NBUF
        for b in range(_NBUF):
            g_copy(t_last + b, b).wait()
            w_copy(t_last + b, b).start()
        for b in range(_NBUF):
            w_copy(t_last + b, b).wait()

    out_t = sc_gather(table_rep, idx_t)
    return out_t.transpose(1, 0, 2)


# speedup vs baseline: 1.2344x; 1.2317x over previous
import functools

import jax
import jax.numpy as jnp
from jax import lax
from jax.experimental import pallas as pl
from jax.experimental.pallas import tpu as pltpu
from jax.experimental.pallas import tpu_sc as plsc

_NUM_CORES = 2
_NUM_SUBCORES = 16
_NW = _NUM_CORES * _NUM_SUBCORES
_BLK = 8
_NBUF = 4
_REP = 4
_SEQ_TC = 10
_G = 512


def _tc_body(idx_ref, hi_ref, lo_ref, _sc_ref, o_ref):
    idx = idx_ref[0, 0, :]
    vocab = hi_ref.shape[0]
    onehot = (idx[:, None] ==
              lax.broadcasted_iota(jnp.int32, (idx.shape[0], vocab), 1))
    oh = onehot.astype(jnp.bfloat16)
    acc = jnp.dot(oh, hi_ref[...], preferred_element_type=jnp.float32)
    acc = acc + jnp.dot(oh, lo_ref[...], preferred_element_type=jnp.float32)
    o_ref[0] = acc


def kernel(prefix, embedding_weight):
    batch, seq = prefix.shape
    vocab, row_dim = embedding_weight.shape
    bpw = batch // _NW
    nh = bpw // _BLK
    seq_sc = seq - _SEQ_TC

    idx_t = prefix.T

    table_rep = jnp.tile(embedding_weight, (_REP, 1))
    rep_off = ((jnp.arange(batch, dtype=jnp.int32) // bpw) % _REP) * vocab
    idx_sc = idx_t + rep_off[None, :]

    mesh = plsc.VectorSubcoreMesh(core_axis_name="c", subcore_axis_name="s")

    @functools.partial(
        pl.kernel,
        mesh=mesh,
        out_type=jax.ShapeDtypeStruct((seq, batch, row_dim), jnp.float32),
        scratch_types=[
            pltpu.VMEM((seq, 128), jnp.int32),
        ] + [pltpu.VMEM((_BLK, row_dim), jnp.float32)] * _NBUF
          + [pltpu.SemaphoreType.DMA] * (2 * _NBUF),
    )
    def sc_gather(table_hbm, idx_hbm, out_hbm, idx_v, *bufs_sems):
        bufs = bufs_sems[:_NBUF]
        gsems = bufs_sems[_NBUF:2 * _NBUF]
        wsems = bufs_sems[2 * _NBUF:]
        wid = lax.axis_index("s") * _NUM_CORES + lax.axis_index("c")
        base = wid * bpw
        pltpu.sync_copy(idx_hbm.at[:, pl.ds((wid // 4) * 128, 128)], idx_v)
        col0 = (wid % 4) * bpw

        n_items = seq_sc * nh

        def g_copy(t, b):
            s, q = _SEQ_TC + t // nh, t % nh
            return pltpu.make_async_copy(
                table_hbm.at[idx_v.at[s, pl.ds(col0 + q * _BLK, _BLK)]],
                bufs[b], gsems[b])

        def w_copy(t, b):
            s, q = _SEQ_TC + t // nh, t % nh
            return pltpu.make_async_copy(
                bufs[b], out_hbm.at[s, pl.ds(base + q * _BLK, _BLK)],
                wsems[b])

        for b in range(_NBUF):
            g_copy(b, b).start()

        @pl.loop(0, n_items - _NBUF, step=_NBUF)
        def _(t):
            for b in range(_NBUF):
                g_copy(t + b, b).wait()
                w_copy(t + b, b).start()
            for b in range(_NBUF):
                w_copy(t + b, b).wait()
                g_copy(t + b + _NBUF, b).start()

        t_last = n_items - _NBUF
        for b in range(_NBUF):
            g_copy(t_last + b, b).wait()
            w_copy(t_last + b, b).start()
        for b in range(_NBUF):
            w_copy(t_last + b, b).wait()

    out_sc = sc_gather(table_rep, idx_sc)

    hi = embedding_weight.astype(jnp.bfloat16)
    lo = (embedding_weight - hi.astype(jnp.float32)).astype(jnp.bfloat16)
    idx_tc = idx_t[:_SEQ_TC].reshape(_SEQ_TC, 1, batch)
    out_t = pl.pallas_call(
        _tc_body,
        grid=(_SEQ_TC, batch // _G),
        in_specs=[
            pl.BlockSpec((1, 1, _G), lambda s, g: (s, 0, g)),
            pl.BlockSpec((vocab, row_dim), lambda s, g: (0, 0)),
            pl.BlockSpec((vocab, row_dim), lambda s, g: (0, 0)),
            pl.BlockSpec(memory_space=pl.ANY),
        ],
        out_specs=pl.BlockSpec((1, _G, row_dim), lambda s, g: (s, g, 0)),
        out_shape=jax.ShapeDtypeStruct((seq, batch, row_dim), jnp.float32),
        input_output_aliases={3: 0},
    )(idx_tc, hi, lo, out_sc)

    return out_t.transpose(1, 0, 2)
